# baseline (speedup 1.0000x reference)
import jax
import jax.numpy as jnp
from jax import lax
from jax.experimental import pallas as pl
from jax.experimental.pallas import tpu as pltpu
from jax.experimental.pallas import tpu_sc as plsc

N = 50000
E = 1600000
G = 64
H = 300
HP = 304
P = 128

NC, NS = 2, 16
PASSES = 6
ROWS_JOB = 4176
ROWS_PAD = 4192
STRIPE = ROWS_PAD // NS
DUMMY_ROW = 4180
K = 64
M = 1536
EPT = E // NS
CHUNK = 2000
NCHUNK = EPT // CHUNK
NVREG = CHUNK // 16
RBLK = 1000


def _sc_agg_body(src_hbm, dst_hbm, h_hbm, out_hbm,
                 sstage0, dstage0, sstage1, dstage1,
                 srcm, dstm, srcidx0, dstidx0, srcidx1, dstidx1,
                 rows0, rows1, esem0, esem1, gsem0, gsem1, agg_sh):
    c = lax.axis_index("c")
    s = lax.axis_index("s")
    r0 = s * STRIPE

    def stage_idx(j, srcidx, dstidx):
        off = j * K
        for t in range(K // 16):
            srcidx[pl.ds(t * 16, 16)] = srcm[pl.ds(off + t * 16, 16)]
            dstidx[pl.ds(t * 16, 16)] = dstm[pl.ds(off + t * 16, 16)]

    def one_batch(j, srcidx, dstidx, rows, gsem):
        stage_idx(j, srcidx, dstidx)
        return pltpu.async_copy(h_hbm.at[srcidx], rows, gsem)

    def drain(cnt):
        nb = cnt // K

        def db(u, _):
            d0 = one_batch(2 * u, srcidx0, dstidx0, rows0, gsem0)
            d1 = one_batch(2 * u + 1, srcidx1, dstidx1, rows1, gsem1)
            d0.wait()
            pltpu.sync_copy(rows0, agg_sh.at[dstidx0], add=True)
            d1.wait()
            pltpu.sync_copy(rows1, agg_sh.at[dstidx1], add=True)
            return 0
        lax.fori_loop(0, nb // 2, db, 0)

        @pl.when(nb % 2 == 1)
        def _():
            d0 = one_batch(nb - 1, srcidx0, dstidx0, rows0, gsem0)
            d0.wait()
            pltpu.sync_copy(rows0, agg_sh.at[dstidx0], add=True)

        @pl.when(cnt > nb * K)
        def _():
            for t in range(K // 16):
                vs = srcm[pl.ds(nb * K + t * 16, 16)]
                vd = dstm[pl.ds(nb * K + t * 16, 16)]
                srcm[pl.ds(t * 16, 16)] = vs
                dstm[pl.ds(t * 16, 16)] = vd
        return cnt - nb * K

    def pass_body(p, _):
        lo = (c * PASSES + p) * ROWS_JOB
        hi = lo + ROWS_JOB

        def zfill(i, _):
            r = i // (HP // 16)
            col = (i % (HP // 16)) * 16
            rows0[r, pl.ds(col, 16)] = jnp.zeros((16,), jnp.float32)
            return 0
        lax.fori_loop(0, K * (HP // 16), zfill, 0)
        for t in range(STRIPE // K):
            pltpu.sync_copy(rows0, agg_sh.at[pl.ds(r0 + t * K, K)])
        pltpu.sync_copy(rows0.at[pl.ds(0, STRIPE % K)],
                        agg_sh.at[pl.ds(r0 + (STRIPE // K) * K, STRIPE % K)])
        plsc.subcore_barrier()

        def scan_chunk(src_st, dst_st, cnt):
            def vb(i, cnt):
                sv = src_st[pl.ds(i * 16, 16)]
                dv = dst_st[pl.ds(i * 16, 16)]
                m = (dv >= lo) & (dv < hi)
                mi = m.astype(jnp.int32)
                csum = plsc.cumsum(mi)
                pos = cnt + csum - 1
                plsc.store_scatter(srcm, [pos], sv, mask=m)
                plsc.store_scatter(dstm, [pos], dv - lo, mask=m)
                cnt = cnt + jnp.squeeze(lax.slice(csum, (15,), (16,)))
                return lax.cond(cnt >= M - 16, drain, lambda t: t, cnt)
            return lax.fori_loop(0, NVREG, vb, cnt)

        def chunk_pair(q, cnt):
            b0 = s * EPT + (2 * q) * CHUNK
            b1 = b0 + CHUNK
            ds0 = pltpu.async_copy(src_hbm.at[pl.ds(b0, CHUNK)], sstage0, esem0)
            dd0 = pltpu.async_copy(dst_hbm.at[pl.ds(b0, CHUNK)], dstage0, esem0)
            ds1 = pltpu.async_copy(src_hbm.at[pl.ds(b1, CHUNK)], sstage1, esem1)
            dd1 = pltpu.async_copy(dst_hbm.at[pl.ds(b1, CHUNK)], dstage1, esem1)
            ds0.wait()
            dd0.wait()
            cnt = scan_chunk(sstage0, dstage0, cnt)
            ds1.wait()
            dd1.wait()
            cnt = scan_chunk(sstage1, dstage1, cnt)
            return cnt

        cnt = lax.fori_loop(0, NCHUNK // 2, chunk_pair, 0)

        for t in range(K // 16):
            srcm[pl.ds(cnt + t * 16, 16)] = jnp.zeros((16,), jnp.int32)
            dstm[pl.ds(cnt + t * 16, 16)] = jnp.full((16,), DUMMY_ROW, jnp.int32)
        cnt = cnt + (K - cnt % K) % K
        drain(cnt)

        plsc.subcore_barrier()
        pltpu.sync_copy(agg_sh.at[pl.ds(r0, STRIPE)],
                        out_hbm.at[c * PASSES + p, pl.ds(r0, STRIPE)])
        return 0

    lax.fori_loop(0, PASSES, pass_body, 0)


def _sc_agg(src, dst, h):
    return pl.kernel(
        _sc_agg_body,
        out_type=jax.ShapeDtypeStruct((NC * PASSES, ROWS_PAD, HP), jnp.float32),
        mesh=plsc.VectorSubcoreMesh(core_axis_name="c", subcore_axis_name="s"),
        compiler_params=pltpu.CompilerParams(
            needs_layout_passes=False,
            use_tc_tiling_on_sc=False,
        ),
        scratch_types=[
            pltpu.VMEM((CHUNK,), jnp.int32),
            pltpu.VMEM((CHUNK,), jnp.int32),
            pltpu.VMEM((CHUNK,), jnp.int32),
            pltpu.VMEM((CHUNK,), jnp.int32),
            pltpu.VMEM((M + K,), jnp.int32),
            pltpu.VMEM((M + K,), jnp.int32),
            pltpu.VMEM((K,), jnp.int32),
            pltpu.VMEM((K,), jnp.int32),
            pltpu.VMEM((K,), jnp.int32),
            pltpu.VMEM((K,), jnp.int32),
            pltpu.VMEM((K, HP), jnp.float32),
            pltpu.VMEM((K, HP), jnp.float32),
            pltpu.SemaphoreType.DMA,
            pltpu.SemaphoreType.DMA,
            pltpu.SemaphoreType.DMA,
            pltpu.SemaphoreType.DMA,
            pltpu.VMEM_SHARED((ROWS_PAD, HP), jnp.float32),
        ],
    )(src, dst, h)


def _mm_relu_body(x_ref, w_ref, b_ref, o_ref):
    o_ref[...] = jnp.maximum(
        jnp.dot(x_ref[...], w_ref[...], preferred_element_type=jnp.float32)
        + b_ref[...][None, :], 0.0)


def _mm_relu(x, W, b):
    din = x.shape[1]
    return pl.pallas_call(
        _mm_relu_body,
        out_shape=jax.ShapeDtypeStruct((N, HP), jnp.float32),
        grid=(N // RBLK,),
        in_specs=[
            pl.BlockSpec((RBLK, din), lambda i: (i, 0)),
            pl.BlockSpec((din, HP), lambda i: (0, 0)),
            pl.BlockSpec((HP,), lambda i: (0,)),
        ],
        out_specs=pl.BlockSpec((RBLK, HP), lambda i: (i, 0)),
    )(x, W, b)


def _combine_body(h_ref, a_ref, w_ref, b_ref, o_ref):
    a = a_ref[...]
    is_ones_col = lax.broadcasted_iota(jnp.int32, (1, HP), 1) == H
    deg = jnp.sum(jnp.where(is_ones_col, a, 0.0), axis=1, keepdims=True)
    u = h_ref[...] + a * (1.0 / jnp.maximum(deg, 1.0))
    o_ref[...] = jnp.maximum(
        jnp.dot(u, w_ref[...], preferred_element_type=jnp.float32)
        + b_ref[...][None, :], 0.0)


def _combine(h, a, W, b):
    return pl.pallas_call(
        _combine_body,
        out_shape=jax.ShapeDtypeStruct((N, HP), jnp.float32),
        grid=(N // RBLK,),
        in_specs=[
            pl.BlockSpec((RBLK, HP), lambda i: (i, 0)),
            pl.BlockSpec((RBLK, HP), lambda i: (i, 0)),
            pl.BlockSpec((HP, HP), lambda i: (0, 0)),
            pl.BlockSpec((HP,), lambda i: (0,)),
        ],
        out_specs=pl.BlockSpec((RBLK, HP), lambda i: (i, 0)),
    )(h, a, W, b)


def _pool_proj_body(h_ref, bt_ref, wp_ref, bp_ref, ne_ref, ge_ref, gsum_ref):
    i = pl.program_id(0)
    h = h_ref[...]
    ne_ref[...] = (jnp.dot(h, wp_ref[...], preferred_element_type=jnp.float32)
                   + bp_ref[...][None, :])
    bt = bt_ref[...].reshape(1, RBLK)
    onehot = (bt == lax.broadcasted_iota(jnp.int32, (G, 1), 0)).astype(jnp.float32)
    part = jnp.dot(onehot, h, preferred_element_type=jnp.float32)

    @pl.when(i == 0)
    def _():
        gsum_ref[...] = part

    @pl.when(i > 0)
    def _():
        gsum_ref[...] += part

    @pl.when(i == pl.num_programs(0) - 1)
    def _():
        gs = gsum_ref[...]
        is_ones_col = lax.broadcasted_iota(jnp.int32, (1, HP), 1) == H
        cntg = jnp.sum(jnp.where(is_ones_col, gs, 0.0), axis=1, keepdims=True)
        inv = 1.0 / jnp.maximum(cntg, 1.0)
        ge_ref[...] = (jnp.dot(gs, wp_ref[...], preferred_element_type=jnp.float32)
                       * inv + bp_ref[...][None, :])


def _pool_proj(h, batch3, Wp, bp):
    return pl.pallas_call(
        _pool_proj_body,
        out_shape=[
            jax.ShapeDtypeStruct((N, P), jnp.float32),
            jax.ShapeDtypeStruct((G, P), jnp.float32),
        ],
        grid=(N // RBLK,),
        in_specs=[
            pl.BlockSpec((RBLK, HP), lambda i: (i, 0)),
            pl.BlockSpec((1, 1, RBLK), lambda i: (i, 0, 0)),
            pl.BlockSpec((HP, P), lambda i: (0, 0)),
            pl.BlockSpec((P,), lambda i: (0,)),
        ],
        out_specs=[
            pl.BlockSpec((RBLK, P), lambda i: (i, 0)),
            pl.BlockSpec((G, P), lambda i: (0, 0)),
        ],
        scratch_shapes=[pltpu.VMEM((G, HP), jnp.float32)],
    )(h, batch3, Wp, bp)


def kernel(x, edge_index, batch, W_in, b_in, W1, b1, W2, b2, W_proj, b_proj):
    src = edge_index[0]
    dst = edge_index[1]
    W_inp = jnp.pad(W_in, ((0, 0), (0, HP - H)))
    b_inp = jnp.pad(b_in, (0, HP - H)).at[H].set(1.0)
    W1p = jnp.pad(W1, ((0, HP - H), (0, HP - H)))
    b1p = jnp.pad(b1, (0, HP - H)).at[H].set(1.0)
    W2p = jnp.pad(W2, ((0, HP - H), (0, HP - H)))
    b2p = jnp.pad(b2, (0, HP - H)).at[H].set(1.0)
    W_projp = jnp.pad(W_proj, ((0, HP - H), (0, 0)))
    batch3 = batch.reshape(N // RBLK, 1, RBLK)

    h0 = _mm_relu(x, W_inp, b_inp)
    agg1 = _sc_agg(src, dst, h0)[:, :ROWS_JOB, :].reshape(-1, HP)[:N]
    h1 = _combine(h0, agg1, W1p, b1p)
    agg2 = _sc_agg(src, dst, h1)[:, :ROWS_JOB, :].reshape(-1, HP)[:N]
    h2 = _combine(h1, agg2, W2p, b2p)
    node_emb, graph_emb = _pool_proj(h2, batch3, W_projp, b_proj)
    return (graph_emb, node_emb)


# speedup vs baseline: 1.0226x; 1.0226x over previous
import jax
import jax.numpy as jnp
from jax import lax
from jax.experimental import pallas as pl
from jax.experimental.pallas import tpu as pltpu
from jax.experimental.pallas import tpu_sc as plsc

N = 50000
E = 1600000
G = 64
H = 300
HP = 304
P = 128

NC, NS = 2, 16
PASSES = 6
ROWS_JOB = 4176
ROWS_PAD = 4192
STRIPE = ROWS_PAD // NS
DUMMY_ROW = 4180
K = 32
M = 1536
EPT = E // NS
CHUNK = 2000
NCHUNK = EPT // CHUNK
NVREG = CHUNK // 16
RBLK = 1000


def _sc_agg_body(src_hbm, dst_hbm, h_hbm, out_hbm,
                 sstage0, dstage0, sstage1, dstage1,
                 srcm, dstm, srcidx0, dstidx0, srcidx1, dstidx1,
                 srcidx2, dstidx2, srcidx3, dstidx3,
                 rows0, rows1, rows2, rows3,
                 esem0, esem1, gsem0, gsem1, gsem2, gsem3, agg_sh):
    c = lax.axis_index("c")
    s = lax.axis_index("s")
    r0 = s * STRIPE

    def stage_idx(j, srcidx, dstidx):
        off = j * K
        for t in range(K // 16):
            srcidx[pl.ds(t * 16, 16)] = srcm[pl.ds(off + t * 16, 16)]
            dstidx[pl.ds(t * 16, 16)] = dstm[pl.ds(off + t * 16, 16)]

    def one_batch(j, srcidx, dstidx, rows, gsem):
        stage_idx(j, srcidx, dstidx)
        return pltpu.async_copy(h_hbm.at[srcidx], rows, gsem)

    def drain(cnt):
        nb = cnt // K

        def db(u, _):
            d0 = one_batch(4 * u, srcidx0, dstidx0, rows0, gsem0)
            d1 = one_batch(4 * u + 1, srcidx1, dstidx1, rows1, gsem1)
            d2 = one_batch(4 * u + 2, srcidx2, dstidx2, rows2, gsem2)
            d3 = one_batch(4 * u + 3, srcidx3, dstidx3, rows3, gsem3)
            d0.wait()
            pltpu.sync_copy(rows0, agg_sh.at[dstidx0], add=True)
            d1.wait()
            pltpu.sync_copy(rows1, agg_sh.at[dstidx1], add=True)
            d2.wait()
            pltpu.sync_copy(rows2, agg_sh.at[dstidx2], add=True)
            d3.wait()
            pltpu.sync_copy(rows3, agg_sh.at[dstidx3], add=True)
            return 0
        lax.fori_loop(0, nb // 4, db, 0)

        def tail(j, _):
            d0 = one_batch(j, srcidx0, dstidx0, rows0, gsem0)
            d0.wait()
            pltpu.sync_copy(rows0, agg_sh.at[dstidx0], add=True)
            return 0
        lax.fori_loop((nb // 4) * 4, nb, tail, 0)

        @pl.when(cnt > nb * K)
        def _():
            for t in range(K // 16):
                vs = srcm[pl.ds(nb * K + t * 16, 16)]
                vd = dstm[pl.ds(nb * K + t * 16, 16)]
                srcm[pl.ds(t * 16, 16)] = vs
                dstm[pl.ds(t * 16, 16)] = vd
        return cnt - nb * K

    def pass_body(p, _):
        lo = (c * PASSES + p) * ROWS_JOB
        hi = lo + ROWS_JOB

        def zfill(i, _):
            r = i // (HP // 16)
            col = (i % (HP // 16)) * 16
            rows0[r, pl.ds(col, 16)] = jnp.zeros((16,), jnp.float32)
            return 0
        lax.fori_loop(0, K * (HP // 16), zfill, 0)
        for t in range(STRIPE // K):
            pltpu.sync_copy(rows0, agg_sh.at[pl.ds(r0 + t * K, K)])
        pltpu.sync_copy(rows0.at[pl.ds(0, STRIPE % K)],
                        agg_sh.at[pl.ds(r0 + (STRIPE // K) * K, STRIPE % K)])
        plsc.subcore_barrier()

        def scan_chunk(src_st, dst_st, cnt):
            def vb(i, cnt):
                sv = src_st[pl.ds(i * 16, 16)]
                dv = dst_st[pl.ds(i * 16, 16)]
                m = (dv >= lo) & (dv < hi)
                mi = m.astype(jnp.int32)
                csum = plsc.cumsum(mi)
                pos = cnt + csum - 1
                plsc.store_scatter(srcm, [pos], sv, mask=m)
                plsc.store_scatter(dstm, [pos], dv - lo, mask=m)
                cnt = cnt + jnp.squeeze(lax.slice(csum, (15,), (16,)))
                return lax.cond(cnt >= M - 16, drain, lambda t: t, cnt)
            return lax.fori_loop(0, NVREG, vb, cnt)

        def chunk_pair(q, cnt):
            b0 = s * EPT + (2 * q) * CHUNK
            b1 = b0 + CHUNK
            ds0 = pltpu.async_copy(src_hbm.at[pl.ds(b0, CHUNK)], sstage0, esem0)
            dd0 = pltpu.async_copy(dst_hbm.at[pl.ds(b0, CHUNK)], dstage0, esem0)
            ds1 = pltpu.async_copy(src_hbm.at[pl.ds(b1, CHUNK)], sstage1, esem1)
            dd1 = pltpu.async_copy(dst_hbm.at[pl.ds(b1, CHUNK)], dstage1, esem1)
            ds0.wait()
            dd0.wait()
            cnt = scan_chunk(sstage0, dstage0, cnt)
            ds1.wait()
            dd1.wait()
            cnt = scan_chunk(sstage1, dstage1, cnt)
            return cnt

        cnt = lax.fori_loop(0, NCHUNK // 2, chunk_pair, 0)

        for t in range(K // 16):
            srcm[pl.ds(cnt + t * 16, 16)] = jnp.zeros((16,), jnp.int32)
            dstm[pl.ds(cnt + t * 16, 16)] = jnp.full((16,), DUMMY_ROW, jnp.int32)
        cnt = cnt + (K - cnt % K) % K
        drain(cnt)

        plsc.subcore_barrier()
        pltpu.sync_copy(agg_sh.at[pl.ds(r0, STRIPE)],
                        out_hbm.at[c * PASSES + p, pl.ds(r0, STRIPE)])
        return 0

    lax.fori_loop(0, PASSES, pass_body, 0)


def _sc_agg(src, dst, h):
    return pl.kernel(
        _sc_agg_body,
        out_type=jax.ShapeDtypeStruct((NC * PASSES, ROWS_PAD, HP), jnp.float32),
        mesh=plsc.VectorSubcoreMesh(core_axis_name="c", subcore_axis_name="s"),
        compiler_params=pltpu.CompilerParams(
            needs_layout_passes=False,
            use_tc_tiling_on_sc=False,
        ),
        scratch_types=[
            pltpu.VMEM((CHUNK,), jnp.int32),
            pltpu.VMEM((CHUNK,), jnp.int32),
            pltpu.VMEM((CHUNK,), jnp.int32),
            pltpu.VMEM((CHUNK,), jnp.int32),
            pltpu.VMEM((M + K,), jnp.int32),
            pltpu.VMEM((M + K,), jnp.int32),
            pltpu.VMEM((K,), jnp.int32),
            pltpu.VMEM((K,), jnp.int32),
            pltpu.VMEM((K,), jnp.int32),
            pltpu.VMEM((K,), jnp.int32),
            pltpu.VMEM((K,), jnp.int32),
            pltpu.VMEM((K,), jnp.int32),
            pltpu.VMEM((K,), jnp.int32),
            pltpu.VMEM((K,), jnp.int32),
            pltpu.VMEM((K, HP), jnp.float32),
            pltpu.VMEM((K, HP), jnp.float32),
            pltpu.VMEM((K, HP), jnp.float32),
            pltpu.VMEM((K, HP), jnp.float32),
            pltpu.SemaphoreType.DMA,
            pltpu.SemaphoreType.DMA,
            pltpu.SemaphoreType.DMA,
            pltpu.SemaphoreType.DMA,
            pltpu.SemaphoreType.DMA,
            pltpu.SemaphoreType.DMA,
            pltpu.VMEM_SHARED((ROWS_PAD, HP), jnp.float32),
        ],
    )(src, dst, h)


def _mm_relu_body(x_ref, w_ref, b_ref, o_ref):
    o_ref[...] = jnp.maximum(
        jnp.dot(x_ref[...], w_ref[...], preferred_element_type=jnp.float32)
        + b_ref[...][None, :], 0.0)


def _mm_relu(x, W, b):
    din = x.shape[1]
    return pl.pallas_call(
        _mm_relu_body,
        out_shape=jax.ShapeDtypeStruct((N, HP), jnp.float32),
        grid=(N // RBLK,),
        in_specs=[
            pl.BlockSpec((RBLK, din), lambda i: (i, 0)),
            pl.BlockSpec((din, HP), lambda i: (0, 0)),
            pl.BlockSpec((HP,), lambda i: (0,)),
        ],
        out_specs=pl.BlockSpec((RBLK, HP), lambda i: (i, 0)),
    )(x, W, b)


def _combine_body(h_ref, a_ref, w_ref, b_ref, o_ref):
    a = a_ref[...]
    is_ones_col = lax.broadcasted_iota(jnp.int32, (1, HP), 1) == H
    deg = jnp.sum(jnp.where(is_ones_col, a, 0.0), axis=1, keepdims=True)
    u = h_ref[...] + a * (1.0 / jnp.maximum(deg, 1.0))
    o_ref[...] = jnp.maximum(
        jnp.dot(u, w_ref[...], preferred_element_type=jnp.float32)
        + b_ref[...][None, :], 0.0)


def _combine(h, a, W, b):
    return pl.pallas_call(
        _combine_body,
        out_shape=jax.ShapeDtypeStruct((N, HP), jnp.float32),
        grid=(N // RBLK,),
        in_specs=[
            pl.BlockSpec((RBLK, HP), lambda i: (i, 0)),
            pl.BlockSpec((RBLK, HP), lambda i: (i, 0)),
            pl.BlockSpec((HP, HP), lambda i: (0, 0)),
            pl.BlockSpec((HP,), lambda i: (0,)),
        ],
        out_specs=pl.BlockSpec((RBLK, HP), lambda i: (i, 0)),
    )(h, a, W, b)


def _pool_proj_body(h_ref, bt_ref, wp_ref, bp_ref, ne_ref, ge_ref, gsum_ref):
    i = pl.program_id(0)
    h = h_ref[...]
    ne_ref[...] = (jnp.dot(h, wp_ref[...], preferred_element_type=jnp.float32)
                   + bp_ref[...][None, :])
    bt = bt_ref[...].reshape(1, RBLK)
    onehot = (bt == lax.broadcasted_iota(jnp.int32, (G, 1), 0)).astype(jnp.float32)
    part = jnp.dot(onehot, h, preferred_element_type=jnp.float32)

    @pl.when(i == 0)
    def _():
        gsum_ref[...] = part

    @pl.when(i > 0)
    def _():
        gsum_ref[...] += part

    @pl.when(i == pl.num_programs(0) - 1)
    def _():
        gs = gsum_ref[...]
        is_ones_col = lax.broadcasted_iota(jnp.int32, (1, HP), 1) == H
        cntg = jnp.sum(jnp.where(is_ones_col, gs, 0.0), axis=1, keepdims=True)
        inv = 1.0 / jnp.maximum(cntg, 1.0)
        ge_ref[...] = (jnp.dot(gs, wp_ref[...], preferred_element_type=jnp.float32)
                       * inv + bp_ref[...][None, :])


def _pool_proj(h, batch3, Wp, bp):
    return pl.pallas_call(
        _pool_proj_body,
        out_shape=[
            jax.ShapeDtypeStruct((N, P), jnp.float32),
            jax.ShapeDtypeStruct((G, P), jnp.float32),
        ],
        grid=(N // RBLK,),
        in_specs=[
            pl.BlockSpec((RBLK, HP), lambda i: (i, 0)),
            pl.BlockSpec((1, 1, RBLK), lambda i: (i, 0, 0)),
            pl.BlockSpec((HP, P), lambda i: (0, 0)),
            pl.BlockSpec((P,), lambda i: (0,)),
        ],
        out_specs=[
            pl.BlockSpec((RBLK, P), lambda i: (i, 0)),
            pl.BlockSpec((G, P), lambda i: (0, 0)),
        ],
        scratch_shapes=[pltpu.VMEM((G, HP), jnp.float32)],
    )(h, batch3, Wp, bp)


def kernel(x, edge_index, batch, W_in, b_in, W1, b1, W2, b2, W_proj, b_proj):
    src = edge_index[0]
    dst = edge_index[1]
    W_inp = jnp.pad(W_in, ((0, 0), (0, HP - H)))
    b_inp = jnp.pad(b_in, (0, HP - H)).at[H].set(1.0)
    W1p = jnp.pad(W1, ((0, HP - H), (0, HP - H)))
    b1p = jnp.pad(b1, (0, HP - H)).at[H].set(1.0)
    W2p = jnp.pad(W2, ((0, HP - H), (0, HP - H)))
    b2p = jnp.pad(b2, (0, HP - H)).at[H].set(1.0)
    W_projp = jnp.pad(W_proj, ((0, HP - H), (0, 0)))
    batch3 = batch.reshape(N // RBLK, 1, RBLK)

    h0 = _mm_relu(x, W_inp, b_inp)
    agg1 = _sc_agg(src, dst, h0)[:, :ROWS_JOB, :].reshape(-1, HP)[:N]
    h1 = _combine(h0, agg1, W1p, b1p)
    agg2 = _sc_agg(src, dst, h1)[:, :ROWS_JOB, :].reshape(-1, HP)[:N]
    h2 = _combine(h1, agg2, W2p, b2p)
    node_emb, graph_emb = _pool_proj(h2, batch3, W_projp, b_proj)
    return (graph_emb, node_emb)


# speedup vs baseline: 1.0281x; 1.0053x over previous
import jax
import jax.numpy as jnp
from jax import lax
from jax.experimental import pallas as pl
from jax.experimental.pallas import tpu as pltpu
from jax.experimental.pallas import tpu_sc as plsc

N = 50000
E = 1600000
G = 64
H = 300
HP = 304
P = 128

NC, NS = 2, 16
PASSES = 6
ROWS_JOB = 4176
ROWS_PAD = 4192
STRIPE = ROWS_PAD // NS
DUMMY_ROW = 4180
K = 32
M = 1536
EPT = E // NS
CHUNK = 2000
NCHUNK = EPT // CHUNK
NVREG = CHUNK // 16
RBLK = 1000


def _sc_agg_body(src_hbm, dst_hbm, h_hbm, out_hbm,
                 sstage0, dstage0, sstage1, dstage1,
                 srcm, srcidx0, dstidx0, srcidx1, dstidx1,
                 srcidx2, dstidx2, srcidx3, dstidx3,
                 rows0, rows1, rows2, rows3,
                 esem0, esem1, gsem0, gsem1, gsem2, gsem3, agg_sh):
    c = lax.axis_index("c")
    s = lax.axis_index("s")
    r0 = s * STRIPE

    def stage_idx(j, srcidx, dstidx):
        off = j * K
        for t in range(K // 16):
            pv = srcm[pl.ds(off + t * 16, 16)]
            srcidx[pl.ds(t * 16, 16)] = pv & 0xFFFF
            dstidx[pl.ds(t * 16, 16)] = lax.shift_right_logical(pv, 16)

    def one_batch(j, srcidx, dstidx, rows, gsem):
        stage_idx(j, srcidx, dstidx)
        return pltpu.async_copy(h_hbm.at[srcidx], rows, gsem)

    def drain(cnt):
        nb = cnt // K

        def db(u, _):
            d0 = one_batch(4 * u, srcidx0, dstidx0, rows0, gsem0)
            d1 = one_batch(4 * u + 1, srcidx1, dstidx1, rows1, gsem1)
            d2 = one_batch(4 * u + 2, srcidx2, dstidx2, rows2, gsem2)
            d3 = one_batch(4 * u + 3, srcidx3, dstidx3, rows3, gsem3)
            d0.wait()
            pltpu.sync_copy(rows0, agg_sh.at[dstidx0], add=True)
            d1.wait()
            pltpu.sync_copy(rows1, agg_sh.at[dstidx1], add=True)
            d2.wait()
            pltpu.sync_copy(rows2, agg_sh.at[dstidx2], add=True)
            d3.wait()
            pltpu.sync_copy(rows3, agg_sh.at[dstidx3], add=True)
            return 0
        lax.fori_loop(0, nb // 4, db, 0)

        def tail(j, _):
            d0 = one_batch(j, srcidx0, dstidx0, rows0, gsem0)
            d0.wait()
            pltpu.sync_copy(rows0, agg_sh.at[dstidx0], add=True)
            return 0
        lax.fori_loop((nb // 4) * 4, nb, tail, 0)

        @pl.when(cnt > nb * K)
        def _():
            for t in range(K // 16):
                vs = srcm[pl.ds(nb * K + t * 16, 16)]
                srcm[pl.ds(t * 16, 16)] = vs
        return cnt - nb * K

    def pass_body(p, _):
        lo = (c * PASSES + p) * ROWS_JOB
        hi = lo + ROWS_JOB

        def zfill(i, _):
            r = i // (HP // 16)
            col = (i % (HP // 16)) * 16
            rows0[r, pl.ds(col, 16)] = jnp.zeros((16,), jnp.float32)
            return 0
        lax.fori_loop(0, K * (HP // 16), zfill, 0)
        for t in range(STRIPE // K):
            pltpu.sync_copy(rows0, agg_sh.at[pl.ds(r0 + t * K, K)])
        pltpu.sync_copy(rows0.at[pl.ds(0, STRIPE % K)],
                        agg_sh.at[pl.ds(r0 + (STRIPE // K) * K, STRIPE % K)])
        plsc.subcore_barrier()

        def scan_chunk(src_st, dst_st, cnt):
            def vb(i, cnt):
                sv = src_st[pl.ds(i * 16, 16)]
                dv = dst_st[pl.ds(i * 16, 16)]
                m = (dv >= lo) & (dv < hi)
                mi = m.astype(jnp.int32)
                csum = plsc.cumsum(mi)
                pos = cnt + csum - 1
                pv = sv | ((dv - lo) << 16)
                plsc.store_scatter(srcm, [pos], pv, mask=m)
                cnt = cnt + jnp.squeeze(lax.slice(csum, (15,), (16,)))
                return lax.cond(cnt >= M - 16, drain, lambda t: t, cnt)
            return lax.fori_loop(0, NVREG, vb, cnt)

        def chunk_pair(q, cnt):
            b0 = s * EPT + (2 * q) * CHUNK
            b1 = b0 + CHUNK
            ds0 = pltpu.async_copy(src_hbm.at[pl.ds(b0, CHUNK)], sstage0, esem0)
            dd0 = pltpu.async_copy(dst_hbm.at[pl.ds(b0, CHUNK)], dstage0, esem0)
            ds1 = pltpu.async_copy(src_hbm.at[pl.ds(b1, CHUNK)], sstage1, esem1)
            dd1 = pltpu.async_copy(dst_hbm.at[pl.ds(b1, CHUNK)], dstage1, esem1)
            ds0.wait()
            dd0.wait()
            cnt = scan_chunk(sstage0, dstage0, cnt)
            ds1.wait()
            dd1.wait()
            cnt = scan_chunk(sstage1, dstage1, cnt)
            return cnt

        cnt = lax.fori_loop(0, NCHUNK // 2, chunk_pair, 0)

        for t in range(K // 16):
            srcm[pl.ds(cnt + t * 16, 16)] = jnp.full((16,), DUMMY_ROW << 16,
                                                     jnp.int32)
        cnt = cnt + (K - cnt % K) % K
        drain(cnt)

        plsc.subcore_barrier()
        pltpu.sync_copy(agg_sh.at[pl.ds(r0, STRIPE)],
                        out_hbm.at[c * PASSES + p, pl.ds(r0, STRIPE)])
        return 0

    lax.fori_loop(0, PASSES, pass_body, 0)


def _sc_agg(src, dst, h):
    return pl.kernel(
        _sc_agg_body,
        out_type=jax.ShapeDtypeStruct((NC * PASSES, ROWS_PAD, HP), jnp.float32),
        mesh=plsc.VectorSubcoreMesh(core_axis_name="c", subcore_axis_name="s"),
        compiler_params=pltpu.CompilerParams(
            needs_layout_passes=False,
            use_tc_tiling_on_sc=False,
        ),
        scratch_types=[
            pltpu.VMEM((CHUNK,), jnp.int32),
            pltpu.VMEM((CHUNK,), jnp.int32),
            pltpu.VMEM((CHUNK,), jnp.int32),
            pltpu.VMEM((CHUNK,), jnp.int32),
            pltpu.VMEM((M + K,), jnp.int32),
            pltpu.VMEM((K,), jnp.int32),
            pltpu.VMEM((K,), jnp.int32),
            pltpu.VMEM((K,), jnp.int32),
            pltpu.VMEM((K,), jnp.int32),
            pltpu.VMEM((K,), jnp.int32),
            pltpu.VMEM((K,), jnp.int32),
            pltpu.VMEM((K,), jnp.int32),
            pltpu.VMEM((K,), jnp.int32),
            pltpu.VMEM((K, HP), jnp.float32),
            pltpu.VMEM((K, HP), jnp.float32),
            pltpu.VMEM((K, HP), jnp.float32),
            pltpu.VMEM((K, HP), jnp.float32),
            pltpu.SemaphoreType.DMA,
            pltpu.SemaphoreType.DMA,
            pltpu.SemaphoreType.DMA,
            pltpu.SemaphoreType.DMA,
            pltpu.SemaphoreType.DMA,
            pltpu.SemaphoreType.DMA,
            pltpu.VMEM_SHARED((ROWS_PAD, HP), jnp.float32),
        ],
    )(src, dst, h)


def _mm_relu_body(x_ref, w_ref, b_ref, o_ref):
    o_ref[...] = jnp.maximum(
        jnp.dot(x_ref[...], w_ref[...], preferred_element_type=jnp.float32)
        + b_ref[...][None, :], 0.0)


def _mm_relu(x, W, b):
    din = x.shape[1]
    return pl.pallas_call(
        _mm_relu_body,
        out_shape=jax.ShapeDtypeStruct((N, HP), jnp.float32),
        grid=(N // RBLK,),
        in_specs=[
            pl.BlockSpec((RBLK, din), lambda i: (i, 0)),
            pl.BlockSpec((din, HP), lambda i: (0, 0)),
            pl.BlockSpec((HP,), lambda i: (0,)),
        ],
        out_specs=pl.BlockSpec((RBLK, HP), lambda i: (i, 0)),
    )(x, W, b)


def _combine_body(h_ref, a_ref, w_ref, b_ref, o_ref):
    a = a_ref[...]
    is_ones_col = lax.broadcasted_iota(jnp.int32, (1, HP), 1) == H
    deg = jnp.sum(jnp.where(is_ones_col, a, 0.0), axis=1, keepdims=True)
    u = h_ref[...] + a * (1.0 / jnp.maximum(deg, 1.0))
    o_ref[...] = jnp.maximum(
        jnp.dot(u, w_ref[...], preferred_element_type=jnp.float32)
        + b_ref[...][None, :], 0.0)


def _combine(h, a, W, b):
    return pl.pallas_call(
        _combine_body,
        out_shape=jax.ShapeDtypeStruct((N, HP), jnp.float32),
        grid=(N // RBLK,),
        in_specs=[
            pl.BlockSpec((RBLK, HP), lambda i: (i, 0)),
            pl.BlockSpec((RBLK, HP), lambda i: (i, 0)),
            pl.BlockSpec((HP, HP), lambda i: (0, 0)),
            pl.BlockSpec((HP,), lambda i: (0,)),
        ],
        out_specs=pl.BlockSpec((RBLK, HP), lambda i: (i, 0)),
    )(h, a, W, b)


def _pool_proj_body(h_ref, bt_ref, wp_ref, bp_ref, ne_ref, ge_ref, gsum_ref):
    i = pl.program_id(0)
    h = h_ref[...]
    ne_ref[...] = (jnp.dot(h, wp_ref[...], preferred_element_type=jnp.float32)
                   + bp_ref[...][None, :])
    bt = bt_ref[...].reshape(1, RBLK)
    onehot = (bt == lax.broadcasted_iota(jnp.int32, (G, 1), 0)).astype(jnp.float32)
    part = jnp.dot(onehot, h, preferred_element_type=jnp.float32)

    @pl.when(i == 0)
    def _():
        gsum_ref[...] = part

    @pl.when(i > 0)
    def _():
        gsum_ref[...] += part

    @pl.when(i == pl.num_programs(0) - 1)
    def _():
        gs = gsum_ref[...]
        is_ones_col = lax.broadcasted_iota(jnp.int32, (1, HP), 1) == H
        cntg = jnp.sum(jnp.where(is_ones_col, gs, 0.0), axis=1, keepdims=True)
        inv = 1.0 / jnp.maximum(cntg, 1.0)
        ge_ref[...] = (jnp.dot(gs, wp_ref[...], preferred_element_type=jnp.float32)
                       * inv + bp_ref[...][None, :])


def _pool_proj(h, batch3, Wp, bp):
    return pl.pallas_call(
        _pool_proj_body,
        out_shape=[
            jax.ShapeDtypeStruct((N, P), jnp.float32),
            jax.ShapeDtypeStruct((G, P), jnp.float32),
        ],
        grid=(N // RBLK,),
        in_specs=[
            pl.BlockSpec((RBLK, HP), lambda i: (i, 0)),
            pl.BlockSpec((1, 1, RBLK), lambda i: (i, 0, 0)),
            pl.BlockSpec((HP, P), lambda i: (0, 0)),
            pl.BlockSpec((P,), lambda i: (0,)),
        ],
        out_specs=[
            pl.BlockSpec((RBLK, P), lambda i: (i, 0)),
            pl.BlockSpec((G, P), lambda i: (0, 0)),
        ],
        scratch_shapes=[pltpu.VMEM((G, HP), jnp.float32)],
    )(h, batch3, Wp, bp)


def kernel(x, edge_index, batch, W_in, b_in, W1, b1, W2, b2, W_proj, b_proj):
    src = edge_index[0]
    dst = edge_index[1]
    W_inp = jnp.pad(W_in, ((0, 0), (0, HP - H)))
    b_inp = jnp.pad(b_in, (0, HP - H)).at[H].set(1.0)
    W1p = jnp.pad(W1, ((0, HP - H), (0, HP - H)))
    b1p = jnp.pad(b1, (0, HP - H)).at[H].set(1.0)
    W2p = jnp.pad(W2, ((0, HP - H), (0, HP - H)))
    b2p = jnp.pad(b2, (0, HP - H)).at[H].set(1.0)
    W_projp = jnp.pad(W_proj, ((0, HP - H), (0, 0)))
    batch3 = batch.reshape(N // RBLK, 1, RBLK)

    h0 = _mm_relu(x, W_inp, b_inp)
    agg1 = _sc_agg(src, dst, h0)[:, :ROWS_JOB, :].reshape(-1, HP)[:N]
    h1 = _combine(h0, agg1, W1p, b1p)
    agg2 = _sc_agg(src, dst, h1)[:, :ROWS_JOB, :].reshape(-1, HP)[:N]
    h2 = _combine(h1, agg2, W2p, b2p)
    node_emb, graph_emb = _pool_proj(h2, batch3, W_projp, b_proj)
    return (graph_emb, node_emb)
